# baseline (speedup 1.0000x reference)
import functools

import numpy as np
import jax
import jax.numpy as jnp
from jax import lax
from jax.experimental import pallas as pl
from jax.experimental.pallas import tpu as pltpu
from jax.experimental.pallas import tpu_sc as plsc

jax.config.update("jax_enable_x64", True)

_RES = 0.3
_B = 10000000
_PMOD = (3856093, 9349669, 3492791)
_N = 262144
_M = 1000000
_CAP = _M + _N
_NW = 32
_CHUNK = _N // _NW
_NV = _CHUNK // 16
_RADIX = 4096
_OUTLEN = 3 * (2 * _CAP + _M)
_ORBASE = 3 * _CAP
_CBASE = 6 * _CAP
_ZBASE = 3 * _M

_c64 = 3.0 * _RES * _RES
_t32 = np.float32(_c64)
if float(_t32) > _c64:
    _t32 = np.nextafter(_t32, np.float32(0.0))
_THR = float(_t32)

_mesh = plsc.VectorSubcoreMesh(core_axis_name="c", subcore_axis_name="s")
_scp = pltpu.CompilerParams(needs_layout_passes=False)
_i32 = jnp.int32
_f32 = jnp.float32


def _wid():
    return lax.axis_index("s") * _i32(2) + lax.axis_index("c")


def _iota():
    return lax.iota(jnp.int32, 16)


def _fori(lo, hi, body, init):
    return lax.fori_loop(_i32(lo) if isinstance(lo, int) else lo,
                         _i32(hi) if isinstance(hi, int) else hi,
                         body, init, unroll=False)


def _floor_div_res(x):
    t0 = x / _f32(_RES)
    ti = t0.astype(jnp.int32)
    tf = ti.astype(jnp.float32)
    return jnp.where(tf > t0, ti - _i32(1), ti)


def _hash16(x, y, z):
    b = _i32(_B)
    acc = None
    for comp, pm in ((x, _PMOD[0]), (y, _PMOD[1]), (z, _PMOD[2])):
        g = _floor_div_res(comp)
        m = lax.rem(g * _i32(pm), b)
        m = jnp.where(m < _i32(0), m + b, m)
        acc = m if acc is None else acc + m
    return lax.rem(acc, b)


def _rsqrt_sum(s):
    i = plsc.bitcast(s, jnp.int32)
    y = plsc.bitcast(_i32(0x5F3759DF) - lax.shift_right_logical(i, _i32(1)),
                     jnp.float32)
    for _ in range(3):
        y = y * (_f32(1.5) - _f32(0.5) * s * y * y)
    return y


def _tc2_body(x, o):
    o[...] = x[...]


def _tc2(buf3d):
    z = np.int32(0)
    return pl.pallas_call(
        _tc2_body,
        grid=(125,),
        in_specs=[pl.BlockSpec((1, 1250, 128), lambda i: (i, np.int32(0), np.int32(0)))],
        out_specs=pl.BlockSpec((1, 1250, 128), lambda i: (i, np.int32(0), np.int32(0))),
        out_shape=jax.ShapeDtypeStruct((125, 1250, 128), jnp.int32),
    )(buf3d)


@functools.partial(
    pl.kernel, mesh=_mesh, compiler_params=_scp,
    scratch_types=[pltpu.VMEM((8192,), jnp.float32),
                   pltpu.VMEM((8192,), jnp.int32),
                   pltpu.VMEM((8192,), jnp.float32)],
)
def _k_base(npf_hbm, pcf_hbm, out_ref, upd_ref, zf, zi, cbuf):
    t = _wid()

    def zero_body(k, _):
        zf[pl.ds(k * _i32(16), 16)] = jnp.zeros((16,), jnp.float32)
        zi[pl.ds(k * _i32(16), 16)] = jnp.zeros((16,), jnp.int32)
        return _i32(0)

    _fori(0, 512, zero_body, _i32(0))

    cp = 8192
    s8 = 93752
    for j in range(12):
        off = jnp.minimum(t * _i32(s8) + _i32(j * cp), _i32(3 * _M - cp))
        pltpu.sync_copy(npf_hbm.at[pl.ds(off, cp)], cbuf)
        pltpu.sync_copy(cbuf, out_ref.at[pl.ds(off, cp)])
    for j in range(12):
        off = jnp.minimum(t * _i32(s8) + _i32(j * cp), _i32(3 * _M - cp))
        pltpu.sync_copy(pcf_hbm.at[pl.ds(off, cp)], cbuf)
        pltpu.sync_copy(cbuf, out_ref.at[pl.ds(_i32(_CBASE) + off, cp)])
    s8z = 142904
    for j in range(18):
        off = jnp.minimum(_i32(_ZBASE) + t * _i32(s8z) + _i32(j * cp),
                          _i32(_CBASE - cp))
        pltpu.sync_copy(zf.at[pl.ds(0, cp)], out_ref.at[pl.ds(off, cp)])
    pltpu.sync_copy(zi.at[pl.ds(0, cp)], upd_ref.at[pl.ds(t * _i32(cp), cp)])


@functools.partial(
    pl.kernel, mesh=_mesh, compiler_params=_scp,
    out_type=(jax.ShapeDtypeStruct((_N,), jnp.int32),
              jax.ShapeDtypeStruct((_NW, _RADIX), jnp.int32)),
    scratch_types=[pltpu.VMEM((3 * _CHUNK,), jnp.float32),
                   pltpu.VMEM((_CHUNK,), jnp.int32),
                   pltpu.VMEM((_RADIX,), jnp.int32)],
)
def _k_hh(pts_hbm, h_o, hist_o, pv, kv, hist):
    t = _wid()
    pltpu.sync_copy(pts_hbm.at[pl.ds(t * _i32(3 * _CHUNK), 3 * _CHUNK)], pv)

    def zb(k, _):
        hist[pl.ds(k * _i32(16), 16)] = jnp.zeros((16,), jnp.int32)
        return _i32(0)

    _fori(0, _RADIX // 16, zb, _i32(0))
    io = _iota()

    def body(j, _):
        lanes = (j * _i32(16) + io) * _i32(3)
        x = plsc.load_gather(pv.at[pl.ds(0, 3 * _CHUNK)], [lanes])
        y = plsc.load_gather(pv.at[pl.ds(0, 3 * _CHUNK)], [lanes + _i32(1)])
        z = plsc.load_gather(pv.at[pl.ds(0, 3 * _CHUNK)], [lanes + _i32(2)])
        h = _hash16(x, y, z)
        kv[pl.ds(j * _i32(16), 16)] = h
        d = h & _i32(_RADIX - 1)
        occ, lastm = plsc.scan_count(d)
        plsc.addupdate_scatter(hist.at[pl.ds(0, _RADIX)], [d], occ,
                               mask=lastm)
        return _i32(0)

    _fori(0, _NV, body, _i32(0))
    pltpu.sync_copy(kv, h_o.at[pl.ds(t * _i32(_CHUNK), _CHUNK)])
    pltpu.sync_copy(hist, hist_o.at[t])


@functools.partial(
    pl.kernel, mesh=_mesh, compiler_params=_scp,
    out_type=jax.ShapeDtypeStruct((_NW, _RADIX), jnp.int32),
    scratch_types=[pltpu.VMEM((_CHUNK,), jnp.int32),
                   pltpu.VMEM((_RADIX,), jnp.int32)],
)
def _k_hist12(keys_hbm, hist_o, kv, hist):
    t = _wid()
    pltpu.sync_copy(keys_hbm.at[pl.ds(t * _i32(_CHUNK), _CHUNK)], kv)

    def zb(k, _):
        hist[pl.ds(k * _i32(16), 16)] = jnp.zeros((16,), jnp.int32)
        return _i32(0)

    _fori(0, _RADIX // 16, zb, _i32(0))

    def body(j, _):
        v = kv[pl.ds(j * _i32(16), 16)]
        d = lax.shift_right_logical(v, _i32(12)) & _i32(_RADIX - 1)
        occ, lastm = plsc.scan_count(d)
        plsc.addupdate_scatter(hist.at[pl.ds(0, _RADIX)], [d], occ,
                               mask=lastm)
        return _i32(0)

    _fori(0, _NV, body, _i32(0))
    pltpu.sync_copy(hist, hist_o.at[t])


@functools.partial(
    pl.kernel, mesh=_mesh, compiler_params=_scp,
    out_type=(jax.ShapeDtypeStruct((_RADIX,), jnp.int32),
              jax.ShapeDtypeStruct((_NW, _RADIX), jnp.int32)),
    scratch_types=[pltpu.VMEM((_NW, 128), jnp.int32),
                   pltpu.VMEM((_NW, 128), jnp.int32),
                   pltpu.VMEM((128,), jnp.int32)],
)
def _k_scan(hist_hbm, tot_o, scan_o, hv, pv, totv):
    w = _wid()
    dbase = w * _i32(128)
    for t in range(_NW):
        pltpu.sync_copy(hist_hbm.at[_i32(t), pl.ds(dbase, 128)],
                        hv.at[_i32(t)])
    for seg in range(8):
        sl = pl.ds(seg * 16, 16)
        acc = jnp.zeros((16,), jnp.int32)
        for t in range(_NW):
            pv[_i32(t), sl] = acc
            acc = acc + hv[_i32(t), sl]
        totv[sl] = acc
    pltpu.sync_copy(totv, tot_o.at[pl.ds(dbase, 128)])
    for t in range(_NW):
        pltpu.sync_copy(pv.at[_i32(t)], scan_o.at[_i32(t), pl.ds(dbase, 128)])


def _make_perm(shift):
    @functools.partial(
        pl.kernel, mesh=_mesh, compiler_params=_scp,
        out_type=(jax.ShapeDtypeStruct((_N,), jnp.int32),
                  jax.ShapeDtypeStruct((_N,), jnp.int32)),
        scratch_types=[pltpu.VMEM((_CHUNK,), jnp.int32),
                       pltpu.VMEM((_CHUNK,), jnp.int32),
                       pltpu.VMEM((_RADIX,), jnp.int32),
                       pltpu.VMEM((_RADIX,), jnp.int32),
                       pltpu.VMEM((64, 128), jnp.int32),
                       pltpu.SemaphoreType.DMA],
    )
    def _k_perm(keys_hbm, vals_hbm, tot_hbm, scan_hbm, ko_o, vo_o,
                kv, vv, tv, noff, dst2, sem):
        t = _wid()
        pltpu.sync_copy(keys_hbm.at[pl.ds(t * _i32(_CHUNK), _CHUNK)], kv)
        pltpu.sync_copy(vals_hbm.at[pl.ds(t * _i32(_CHUNK), _CHUNK)], vv)
        pltpu.sync_copy(tot_hbm, tv)
        pltpu.sync_copy(scan_hbm.at[t], noff)

        def scan_body(k, carry):
            sl = pl.ds(k * _i32(16), 16)
            v = tv[sl]
            c = plsc.cumsum(v)
            noff[sl] = noff[sl] + (c - v) + carry
            return carry + c[15]

        _fori(0, _RADIX // 16, scan_body, _i32(0))

        io = _iota()

        def body(j, _):
            v = kv[pl.ds(j * _i32(16), 16)]
            d = lax.shift_right_logical(v, _i32(shift)) & _i32(_RADIX - 1)
            occ, lastm = plsc.scan_count(d)
            cur = plsc.load_gather(noff.at[pl.ds(0, _RADIX)], [d])
            dstv = cur + occ - _i32(1)
            plsc.store_scatter(noff.at[pl.ds(0, _RADIX)], [d], cur + occ,
                               mask=lastm)
            rows = jnp.zeros((16,), jnp.int32) + j // _i32(8)
            cols = (j % _i32(8)) * _i32(16) + io
            plsc.store_scatter(dst2.at[:, :], [rows, cols], dstv)
            return _i32(0)

        _fori(0, _NV, body, _i32(0))

        hs = []
        for c in range(64):
            hs.append(pltpu.async_copy(
                kv.at[pl.ds(c * 128, 128)], ko_o.at[dst2.at[_i32(c)]], sem))
            hs.append(pltpu.async_copy(
                vv.at[pl.ds(c * 128, 128)], vo_o.at[dst2.at[_i32(c)]], sem))
            if c % 8 == 7:
                for h in hs:
                    h.wait()
                hs = []

    return _k_perm


_perm0 = _make_perm(0)
_perm12 = _make_perm(12)


@functools.partial(
    pl.kernel, mesh=_mesh, compiler_params=_scp,
    out_type=(jax.ShapeDtypeStruct((_NW, _CHUNK), jnp.int32),
              jax.ShapeDtypeStruct((_NW, _CHUNK), jnp.int32),
              jax.ShapeDtypeStruct((_NW, 16), jnp.int32)),
    scratch_types=[pltpu.VMEM((_CHUNK,), jnp.int32),
                   pltpu.VMEM((_CHUNK,), jnp.int32),
                   pltpu.VMEM((_CHUNK + 16,), jnp.int32),
                   pltpu.VMEM((_CHUNK + 16,), jnp.int32),
                   pltpu.VMEM((16,), jnp.int32),
                   pltpu.VMEM((16,), jnp.int32)],
)
def _k_win(ks_hbm, vs_hbm, wi_o, wh_o, wcnt_o, kv, vv, wiv, whv, pb, t16):
    t = _wid()
    pltpu.sync_copy(ks_hbm.at[pl.ds(t * _i32(_CHUNK), _CHUNK)], kv)
    pltpu.sync_copy(vs_hbm.at[pl.ds(t * _i32(_CHUNK), _CHUNK)], vv)

    @pl.when(t > 0)
    def _():
        pltpu.sync_copy(ks_hbm.at[pl.ds(t * _i32(_CHUNK) - 16, 16)], pb)

    @pl.when(t == 0)
    def _():
        pb[...] = jnp.full((16,), -1, jnp.int32)

    io = _iota()
    carry0 = pb[...][15]

    def body(j, carry):
        prevlast, wcount = carry
        sl = pl.ds(j * _i32(16), 16)
        v = kv[sl]
        t16[...] = v
        shv = plsc.load_gather(t16.at[pl.ds(0, 16)],
                               [jnp.maximum(io - _i32(1), _i32(0))])
        shv = jnp.where(io == _i32(0), prevlast, shv)
        f = v != shv
        plsc.store_compressed(whv.at[pl.ds(wcount, 16)], v, mask=f)
        plsc.store_compressed(wiv.at[pl.ds(wcount, 16)], vv[sl], mask=f)
        pc = plsc.all_reduce_population_count(f)[0]
        return v[15], wcount + pc

    _, wcount = _fori(0, _NV, body, (carry0, _i32(0)))
    pltpu.sync_copy(wiv.at[pl.ds(0, _CHUNK)], wi_o.at[t])
    pltpu.sync_copy(whv.at[pl.ds(0, _CHUNK)], wh_o.at[t])
    t16[...] = jnp.zeros((16,), jnp.int32) + wcount
    pltpu.sync_copy(t16, wcnt_o.at[t])


@functools.partial(
    pl.kernel, mesh=_mesh, compiler_params=_scp,
    out_type=jax.ShapeDtypeStruct((_NW, 16), jnp.int32),
    scratch_types=[pltpu.VMEM((_CHUNK,), jnp.int32),
                   pltpu.VMEM((_CHUNK,), jnp.int32),
                   pltpu.VMEM((128,), jnp.int32),
                   pltpu.VMEM((128,), jnp.int32),
                   pltpu.VMEM((128,), jnp.int32),
                   pltpu.VMEM((128,), jnp.int32),
                   pltpu.VMEM((128,), jnp.int32),
                   pltpu.VMEM((128,), jnp.float32),
                   pltpu.VMEM((128,), jnp.float32),
                   pltpu.VMEM((128,), jnp.float32),
                   pltpu.VMEM((128,), jnp.float32),
                   pltpu.VMEM((128,), jnp.float32),
                   pltpu.VMEM((128,), jnp.float32),
                   pltpu.VMEM((128,), jnp.float32),
                   pltpu.VMEM((128,), jnp.float32),
                   pltpu.VMEM((128,), jnp.float32),
                   pltpu.VMEM((128,), jnp.int32),
                   pltpu.VMEM((128,), jnp.int32),
                   pltpu.VMEM((128,), jnp.int32),
                   pltpu.VMEM((_CHUNK + 16,), jnp.int32),
                   pltpu.VMEM((_CHUNK + 16,), jnp.float32),
                   pltpu.VMEM((_CHUNK + 16,), jnp.float32),
                   pltpu.VMEM((_CHUNK + 16,), jnp.float32),
                   pltpu.VMEM((16,), jnp.int32),
                   pltpu.VMEM((16,), jnp.float32),
                   pltpu.SemaphoreType.DMA],
)
def _k_updmask(wi_hbm, wh_hbm, wcnt_hbm, buf_hbm, npf_hbm, vcm_hbm,
               ptsf_hbm, colf_hbm, upd_ref, out_ref, wupd_o,
               wiv, whv, bidx, lov, sidx, wic, gidx, gx, gy, gz,
               px, py, pz, cxv, cyv, czv, gv,
               uidx, udat, csafe, ccx, ccy, ccz, i16, f16, sem):
    t = _wid()
    bsrc = buf_hbm
    pltpu.sync_copy(wcnt_hbm.at[t], i16)
    nw = i16[...][0]
    pltpu.sync_copy(wi_hbm.at[t], wiv)
    pltpu.sync_copy(wh_hbm.at[t], whv)
    io = _iota()
    nc = (nw + _i32(127)) // _i32(128)

    def chunk_body(c, ccnt):
        cb = c * _i32(128)

        def bi(j, _):
            sl = pl.ds(j * _i32(16), 16)
            wh = jnp.clip(whv[pl.ds(cb + j * _i32(16), 16)],
                          _i32(0), _i32(_B - 1))
            wival = jnp.clip(wiv[pl.ds(cb + j * _i32(16), 16)],
                             _i32(0), _i32(_N - 1))
            bidx[sl] = wh * _i32(2)
            wic[sl] = wival
            return _i32(0)

        _fori(0, 8, bi, _i32(0))
        pltpu.async_copy(bsrc.at[bidx], lov, sem).wait()

        def si(j, _):
            sl = pl.ds(j * _i32(16), 16)
            sidx[sl] = jnp.clip(lov[sl], _i32(0), _i32(_M - 1))
            return _i32(0)

        _fori(0, 8, si, _i32(0))
        pltpu.async_copy(vcm_hbm.at[sidx], gv, sem).wait()
        for c3, dst in ((0, gx), (1, gy), (2, gz)):
            def mk(j, _, c3=c3):
                sl = pl.ds(j * _i32(16), 16)
                gidx[sl] = sidx[sl] * _i32(3) + _i32(c3)
                return _i32(0)

            _fori(0, 8, mk, _i32(0))
            pltpu.async_copy(npf_hbm.at[gidx], dst, sem).wait()
        for c3, dst, dst2 in ((0, px, cxv), (1, py, cyv), (2, pz, czv)):
            def mk2(j, _, c3=c3):
                sl = pl.ds(j * _i32(16), 16)
                gidx[sl] = wic[sl] * _i32(3) + _i32(c3)
                return _i32(0)

            _fori(0, 8, mk2, _i32(0))
            pltpu.async_copy(ptsf_hbm.at[gidx], dst, sem).wait()
            pltpu.async_copy(colf_hbm.at[gidx], dst2, sem).wait()

        def cv(j, ccnt):
            sl = pl.ds(j * _i32(16), 16)
            valid = (cb + j * _i32(16) + io) < nw
            lo = lov[sl]
            dx = gx[sl] - px[sl]
            dy = gy[sl] - py[sl]
            dz = gz[sl] - pz[sl]
            d2 = (dx * dx + dy * dy) + dz * dz
            updv = ((lo == _i32(-1)) | (d2 > _f32(_THR)))
            updv = jnp.where(valid, updv.astype(jnp.int32), _i32(0))
            cmask = (lo > _i32(-1)) & (gv[sl] == _i32(0)) & valid
            wival = wic[sl]
            uidx[sl] = wival
            udat[sl] = updv
            plsc.store_compressed(csafe.at[pl.ds(ccnt, 16)], sidx[sl],
                                  mask=cmask)
            plsc.store_compressed(ccx.at[pl.ds(ccnt, 16)], cxv[sl], mask=cmask)
            plsc.store_compressed(ccy.at[pl.ds(ccnt, 16)], cyv[sl], mask=cmask)
            plsc.store_compressed(ccz.at[pl.ds(ccnt, 16)], czv[sl], mask=cmask)
            pc = plsc.all_reduce_population_count(cmask)[0]
            return ccnt + pc

        ccnt = _fori(0, 8, cv, ccnt)

        u0 = uidx[pl.ds(0, 16)][0]
        d0 = udat[pl.ds(0, 16)][0]

        def fixv(j, _):
            sl = pl.ds(j * _i32(16), 16)
            valid = (cb + j * _i32(16) + io) < nw
            uidx[sl] = jnp.where(valid, uidx[sl], u0)
            udat[sl] = jnp.where(valid, udat[sl], d0)
            return _i32(0)

        _fori(0, 8, fixv, _i32(0))
        pltpu.async_copy(udat, upd_ref.at[uidx], sem).wait()
        return ccnt

    ccnt = _fori(0, nc, chunk_body, _i32(0))
    ncv = (ccnt + _i32(15)) // _i32(16)

    def col_body(j, _):
        sl = pl.ds(j * _i32(16), 16)
        valid = (j * _i32(16) + io) < ccnt
        s = csafe[sl]
        s = jnp.where(valid, s, s[0])
        vx = ccx[sl]
        vy = ccy[sl]
        vz = ccz[sl]
        base = _i32(_CBASE) + s * _i32(3)
        i16[...] = base
        f16[...] = jnp.where(valid, vx, vx[0])
        pltpu.async_copy(f16, out_ref.at[i16], sem).wait()
        i16[...] = base + _i32(1)
        f16[...] = jnp.where(valid, vy, vy[0])
        pltpu.async_copy(f16, out_ref.at[i16], sem).wait()
        i16[...] = base + _i32(2)
        f16[...] = jnp.where(valid, vz, vz[0])
        pltpu.async_copy(f16, out_ref.at[i16], sem).wait()
        return _i32(0)

    _fori(0, ncv, col_body, _i32(0))
    i16[...] = jnp.zeros((16,), jnp.int32)
    pltpu.sync_copy(i16, wupd_o.at[t])


@functools.partial(
    pl.kernel, mesh=_mesh, compiler_params=_scp,
    out_type=jax.ShapeDtypeStruct((_NW, 16), jnp.int32),
    scratch_types=[pltpu.VMEM((_CHUNK,), jnp.int32),
                   pltpu.VMEM((16,), jnp.int32)],
)
def _k_cnt(upd_ref, cnt_o, uv, t16):
    t = _wid()
    pltpu.sync_copy(upd_ref.at[pl.ds(t * _i32(_CHUNK), _CHUNK)], uv)

    def body(j, s):
        v = uv[pl.ds(j * _i32(16), 16)]
        return s + plsc.cumsum(v)[15]

    s = _fori(0, _NV, body, _i32(0))
    t16[...] = jnp.zeros((16,), jnp.int32) + s
    pltpu.sync_copy(t16, cnt_o.at[t])


@functools.partial(
    pl.kernel, mesh=_mesh, compiler_params=_scp,
    out_type=jax.ShapeDtypeStruct((_NW, 16), jnp.int32),
    scratch_types=[pltpu.VMEM((512,), jnp.int32),
                   pltpu.VMEM((2048,), jnp.int32),
                   pltpu.VMEM((2048,), jnp.int32),
                   pltpu.VMEM((3 * 2048,), jnp.float32),
                   pltpu.VMEM((3 * 2048,), jnp.float32),
                   pltpu.VMEM((_CHUNK + 16,), jnp.int32),
                   pltpu.VMEM((_CHUNK + 16,), jnp.int32),
                   pltpu.VMEM((_CHUNK + 16,), jnp.float32),
                   pltpu.VMEM((_CHUNK + 16,), jnp.float32),
                   pltpu.VMEM((_CHUNK + 16,), jnp.float32),
                   pltpu.VMEM((_CHUNK + 16,), jnp.float32),
                   pltpu.VMEM((_CHUNK + 16,), jnp.float32),
                   pltpu.VMEM((_CHUNK + 16,), jnp.float32),
                   pltpu.VMEM((128,), jnp.int32),
                   pltpu.VMEM((128,), jnp.int32),
                   pltpu.VMEM((16,), jnp.int32),
                   pltpu.SemaphoreType.DMA],
)
def _k_scat(ucntf_hbm, hf_hbm, pts3_hbm, nrm3_hbm, upd_ref, out_ref,
            buf_ref, done_o,
            ucv, uv, hv, ptv, nmv,
            cslot, chh, csx, csy, csz, csnx, csny, csnz,
            idxb, zi128, t16, sem):
    t = _wid()
    bdst = buf_ref
    pltpu.sync_copy(ucntf_hbm, ucv)
    io = _iota()
    g1 = plsc.load_gather(ucv.at[pl.ds(0, 512)], [io * _i32(16)])
    g2 = plsc.load_gather(ucv.at[pl.ds(0, 512)], [io * _i32(16) + _i32(256)])
    base = (plsc.cumsum(jnp.where(io < t, g1, _i32(0)))[15]
            + plsc.cumsum(jnp.where(io + _i32(16) < t, g2, _i32(0)))[15])
    for k in range(8):
        zi128[pl.ds(k * 16, 16)] = jnp.zeros((16,), jnp.int32)

    def sub(sb, cnt):
        off = t * _i32(_CHUNK) + sb * _i32(2048)
        pltpu.sync_copy(upd_ref.at[pl.ds(off, 2048)], uv)
        pltpu.sync_copy(hf_hbm.at[pl.ds(off, 2048)], hv)
        pltpu.sync_copy(pts3_hbm.at[pl.ds(off * _i32(3), 3 * 2048)], ptv)
        pltpu.sync_copy(nrm3_hbm.at[pl.ds(off * _i32(3), 3 * 2048)], nmv)

        def vec(j, cnt):
            sl = pl.ds(j * _i32(16), 16)
            u = uv[sl]
            ub = u > _i32(0)
            pfx = plsc.cumsum(u)
            slot = _i32(_M) + base + cnt + pfx - _i32(1)
            lanes = (j * _i32(16) + io) * _i32(3)
            x = plsc.load_gather(ptv.at[pl.ds(0, 3 * 2048)], [lanes])
            y = plsc.load_gather(ptv.at[pl.ds(0, 3 * 2048)],
                                 [lanes + _i32(1)])
            z = plsc.load_gather(ptv.at[pl.ds(0, 3 * 2048)],
                                 [lanes + _i32(2)])
            nx = plsc.load_gather(nmv.at[pl.ds(0, 3 * 2048)], [lanes])
            ny = plsc.load_gather(nmv.at[pl.ds(0, 3 * 2048)],
                                  [lanes + _i32(1)])
            nz = plsc.load_gather(nmv.at[pl.ds(0, 3 * 2048)],
                                  [lanes + _i32(2)])
            s = nx * nx + ny * ny + nz * nz
            r = s * _rsqrt_sum(s) + _f32(1e-8)
            nx = nx / r
            ny = ny / r
            nz = nz / r
            plsc.store_compressed(cslot.at[pl.ds(cnt, 16)], slot, mask=ub)
            plsc.store_compressed(chh.at[pl.ds(cnt, 16)], hv[sl], mask=ub)
            plsc.store_compressed(csx.at[pl.ds(cnt, 16)], x, mask=ub)
            plsc.store_compressed(csy.at[pl.ds(cnt, 16)], y, mask=ub)
            plsc.store_compressed(csz.at[pl.ds(cnt, 16)], z, mask=ub)
            plsc.store_compressed(csnx.at[pl.ds(cnt, 16)], nx, mask=ub)
            plsc.store_compressed(csny.at[pl.ds(cnt, 16)], ny, mask=ub)
            plsc.store_compressed(csnz.at[pl.ds(cnt, 16)], nz, mask=ub)
            return cnt + pfx[15]

        return _fori(0, 128, vec, cnt)

    cnt = _fori(0, 4, sub, _i32(0))

    @pl.when(cnt > 0)
    def _():
        d_slot = cslot[pl.ds(0, 16)][0]
        d_h = chh[pl.ds(0, 16)][0]
        d_x = csx[pl.ds(0, 16)][0]
        d_y = csy[pl.ds(0, 16)][0]
        d_z = csz[pl.ds(0, 16)][0]
        d_nx = csnx[pl.ds(0, 16)][0]
        d_ny = csny[pl.ds(0, 16)][0]
        d_nz = csnz[pl.ds(0, 16)][0]
        padend = ((cnt + _i32(127)) // _i32(128)) * _i32(128)

        def fill(j, _):
            sl = pl.ds(j * _i32(16), 16)
            valid = (j * _i32(16) + io) < cnt
            cslot[sl] = jnp.where(valid, cslot[sl], d_slot)
            chh[sl] = jnp.where(valid, chh[sl], d_h)
            csx[sl] = jnp.where(valid, csx[sl], d_x)
            csy[sl] = jnp.where(valid, csy[sl], d_y)
            csz[sl] = jnp.where(valid, csz[sl], d_z)
            csnx[sl] = jnp.where(valid, csnx[sl], d_nx)
            csny[sl] = jnp.where(valid, csny[sl], d_ny)
            csnz[sl] = jnp.where(valid, csnz[sl], d_nz)
            return _i32(0)

        _fori(cnt // _i32(16), padend // _i32(16), fill, _i32(0))
        nch = (cnt + _i32(127)) // _i32(128)

        def sc(c, _):
            cb = c * _i32(128)
            srcs = (csx, csy, csz, csnx, csny, csnz)
            for c3 in range(3):
                def mkidx(j, _2, c3=c3):
                    sl = pl.ds(j * _i32(16), 16)
                    s = cslot[pl.ds(cb + j * _i32(16), 16)]
                    idxb[sl] = s * _i32(3) + _i32(c3)
                    return _i32(0)

                _fori(0, 8, mkidx, _i32(0))
                pltpu.async_copy(srcs[c3].at[pl.ds(cb, 128)],
                                 out_ref.at[idxb], sem).wait()

                def mkidx2(j, _2, c3=c3):
                    sl = pl.ds(j * _i32(16), 16)
                    s = cslot[pl.ds(cb + j * _i32(16), 16)]
                    idxb[sl] = _i32(_ORBASE) + s * _i32(3) + _i32(c3)
                    return _i32(0)

                _fori(0, 8, mkidx2, _i32(0))
                pltpu.async_copy(srcs[3 + c3].at[pl.ds(cb, 128)],
                                 out_ref.at[idxb], sem).wait()

            def mkidxb(j, _2):
                sl = pl.ds(j * _i32(16), 16)
                hh = chh[pl.ds(cb + j * _i32(16), 16)]
                idxb[sl] = hh * _i32(2)
                return _i32(0)

            _fori(0, 8, mkidxb, _i32(0))
            pltpu.async_copy(cslot.at[pl.ds(cb, 128)],
                             bdst.at[idxb], sem).wait()

            def mkidxb2(j, _2):
                sl = pl.ds(j * _i32(16), 16)
                hh = chh[pl.ds(cb + j * _i32(16), 16)]
                idxb[sl] = hh * _i32(2) + _i32(1)
                return _i32(0)

            _fori(0, 8, mkidxb2, _i32(0))
            pltpu.async_copy(zi128.at[pl.ds(0, 128)],
                             bdst.at[idxb], sem).wait()
            return _i32(0)

        _fori(0, nch, sc, _i32(0))

    t16[...] = jnp.zeros((16,), jnp.int32)
    pltpu.sync_copy(t16, done_o.at[t])


def kernel(points, colors, normals, buffer_pt_index, neural_points,
           point_colors, valid_color_mask, point_ts_update, travel_dist,
           cur_ts):
    ptsf = points.reshape(3 * _N)
    nrmf = normals.reshape(3 * _N)

    hf, hist1 = _k_hh(ptsf)
    tot1, scan1 = _k_scan(hist1)
    v0 = jnp.arange(_N, dtype=jnp.int32)
    k1, v1 = _perm0(hf, v0, tot1, scan1)
    hist2 = _k_hist12(k1)
    tot2, scan2 = _k_scan(hist2)
    k2, v2 = _perm12(k1, v1, tot2, scan2)
    wi, wh, wcnt = _k_win(k2, v2)

    out_r = jax.empty_ref(jax.ShapeDtypeStruct((_OUTLEN,), jnp.float32))
    upd_r = jax.empty_ref(jax.ShapeDtypeStruct((_N,), jnp.int32))
    _k_base(neural_points.reshape(3 * _M), point_colors.reshape(3 * _M),
            out_r, upd_r)

    bufflat = lax.bitcast_convert_type(buffer_pt_index,
                                       jnp.int32).reshape(2 * _B)
    _k_updmask(wi, wh, wcnt, bufflat, neural_points.reshape(3 * _M),
               valid_color_mask, ptsf, colors.reshape(3 * _N), upd_r, out_r)

    ucnt = _k_cnt(upd_r)

    bufcp = _tc2(bufflat.reshape(125, 1250, 128))
    buf_r = jax.new_ref(bufcp.reshape(2 * _B))
    _k_scat(ucnt.reshape(512), hf, ptsf, nrmf, upd_r, out_r, buf_r)

    out = out_r[...].reshape(2 * _CAP + _M, 3)
    buffer_new = lax.bitcast_convert_type(
        buf_r[...].reshape(_B, 2), jnp.int64)
    return out, buffer_new


# speedup vs baseline: 1.0008x; 1.0008x over previous
import functools

import numpy as np
import jax
import jax.numpy as jnp
from jax import lax
from jax.experimental import pallas as pl
from jax.experimental.pallas import tpu as pltpu
from jax.experimental.pallas import tpu_sc as plsc

jax.config.update("jax_enable_x64", True)

_RES = 0.3
_B = 10000000
_PMOD = (3856093, 9349669, 3492791)
_N = 262144
_M = 1000000
_CAP = _M + _N
_NW = 32
_CHUNK = _N // _NW
_NV = _CHUNK // 16
_RADIX = 4096
_OUTLEN = 3 * (2 * _CAP + _M)
_ORBASE = 3 * _CAP
_CBASE = 6 * _CAP
_ZBASE = 3 * _M

_c64 = 3.0 * _RES * _RES
_t32 = np.float32(_c64)
if float(_t32) > _c64:
    _t32 = np.nextafter(_t32, np.float32(0.0))
_THR = float(_t32)

_mesh = plsc.VectorSubcoreMesh(core_axis_name="c", subcore_axis_name="s")
_scp = pltpu.CompilerParams(needs_layout_passes=False)
_i32 = jnp.int32
_f32 = jnp.float32


def _wid():
    return lax.axis_index("s") * _i32(2) + lax.axis_index("c")


def _iota():
    return lax.iota(jnp.int32, 16)


def _fori(lo, hi, body, init):
    return lax.fori_loop(_i32(lo) if isinstance(lo, int) else lo,
                         _i32(hi) if isinstance(hi, int) else hi,
                         body, init, unroll=False)


def _floor_div_res(x):
    t0 = x / _f32(_RES)
    ti = t0.astype(jnp.int32)
    tf = ti.astype(jnp.float32)
    return jnp.where(tf > t0, ti - _i32(1), ti)


def _hash16(x, y, z):
    b = _i32(_B)
    acc = None
    for comp, pm in ((x, _PMOD[0]), (y, _PMOD[1]), (z, _PMOD[2])):
        g = _floor_div_res(comp)
        m = lax.rem(g * _i32(pm), b)
        m = jnp.where(m < _i32(0), m + b, m)
        acc = m if acc is None else acc + m
    return lax.rem(acc, b)


def _rsqrt_sum(s):
    i = plsc.bitcast(s, jnp.int32)
    y = plsc.bitcast(_i32(0x5F3759DF) - lax.shift_right_logical(i, _i32(1)),
                     jnp.float32)
    for _ in range(3):
        y = y * (_f32(1.5) - _f32(0.5) * s * y * y)
    return y


def _tc2_body(x, o):
    o[...] = x[...]


def _tc2(buf3d):
    z = np.int32(0)
    return pl.pallas_call(
        _tc2_body,
        grid=(125,),
        in_specs=[pl.BlockSpec((1, 1250, 128), lambda i: (i, np.int32(0), np.int32(0)))],
        out_specs=pl.BlockSpec((1, 1250, 128), lambda i: (i, np.int32(0), np.int32(0))),
        out_shape=jax.ShapeDtypeStruct((125, 1250, 128), jnp.int32),
    )(buf3d)


@functools.partial(
    pl.kernel, mesh=_mesh, compiler_params=_scp,
    scratch_types=[pltpu.VMEM((8192,), jnp.float32),
                   pltpu.VMEM((8192,), jnp.int32),
                   pltpu.VMEM((32768,), jnp.float32),
                   pltpu.VMEM((32768,), jnp.float32),
                   pltpu.SemaphoreType.DMA],
)
def _k_base(npf_hbm, pcf_hbm, out_ref, upd_ref, zf, zi, nbuf, pbuf, sem):
    t = _wid()

    def zero_body(k, _):
        zf[pl.ds(k * _i32(16), 16)] = jnp.zeros((16,), jnp.float32)
        zi[pl.ds(k * _i32(16), 16)] = jnp.zeros((16,), jnp.int32)
        return _i32(0)

    _fori(0, 512, zero_body, _i32(0))

    cp = 8192
    s8 = 93752
    s8z = 142904
    zs = []
    for j in range(18):
        off = jnp.minimum(_i32(_ZBASE) + t * _i32(s8z) + _i32(j * cp),
                          _i32(_CBASE - cp))
        zs.append(pltpu.async_copy(zf.at[pl.ds(0, cp)],
                                   out_ref.at[pl.ds(off, cp)], sem))
    zs.append(pltpu.async_copy(zi.at[pl.ds(0, cp)],
                               upd_ref.at[pl.ds(t * _i32(cp), cp)], sem))
    ws = []
    for w in range(3):
        for h in ws:
            h.wait()
        ws = []
        hs = []
        for k in range(4):
            j = 4 * w + k
            off = jnp.minimum(t * _i32(s8) + _i32(j * cp),
                              _i32(3 * _M - cp))
            hs.append(pltpu.async_copy(npf_hbm.at[pl.ds(off, cp)],
                                       nbuf.at[pl.ds(k * cp, cp)], sem))
            hs.append(pltpu.async_copy(pcf_hbm.at[pl.ds(off, cp)],
                                       pbuf.at[pl.ds(k * cp, cp)], sem))
        for h in hs:
            h.wait()
        for k in range(4):
            j = 4 * w + k
            off = jnp.minimum(t * _i32(s8) + _i32(j * cp),
                              _i32(3 * _M - cp))
            ws.append(pltpu.async_copy(nbuf.at[pl.ds(k * cp, cp)],
                                       out_ref.at[pl.ds(off, cp)], sem))
            ws.append(pltpu.async_copy(pbuf.at[pl.ds(k * cp, cp)],
                                       out_ref.at[pl.ds(_i32(_CBASE) + off,
                                                        cp)], sem))
    for h in ws:
        h.wait()
    for h in zs:
        h.wait()


@functools.partial(
    pl.kernel, mesh=_mesh, compiler_params=_scp,
    out_type=(jax.ShapeDtypeStruct((_N,), jnp.int32),
              jax.ShapeDtypeStruct((_NW, _RADIX), jnp.int32)),
    scratch_types=[pltpu.VMEM((3 * _CHUNK,), jnp.float32),
                   pltpu.VMEM((_CHUNK,), jnp.int32),
                   pltpu.VMEM((_RADIX,), jnp.int32)],
)
def _k_hh(pts_hbm, h_o, hist_o, pv, kv, hist):
    t = _wid()
    pltpu.sync_copy(pts_hbm.at[pl.ds(t * _i32(3 * _CHUNK), 3 * _CHUNK)], pv)

    def zb(k, _):
        hist[pl.ds(k * _i32(16), 16)] = jnp.zeros((16,), jnp.int32)
        return _i32(0)

    _fori(0, _RADIX // 16, zb, _i32(0))
    io = _iota()

    def body(j, _):
        lanes = (j * _i32(16) + io) * _i32(3)
        x = plsc.load_gather(pv.at[pl.ds(0, 3 * _CHUNK)], [lanes])
        y = plsc.load_gather(pv.at[pl.ds(0, 3 * _CHUNK)], [lanes + _i32(1)])
        z = plsc.load_gather(pv.at[pl.ds(0, 3 * _CHUNK)], [lanes + _i32(2)])
        h = _hash16(x, y, z)
        kv[pl.ds(j * _i32(16), 16)] = h
        d = h & _i32(_RADIX - 1)
        occ, lastm = plsc.scan_count(d)
        plsc.addupdate_scatter(hist.at[pl.ds(0, _RADIX)], [d], occ,
                               mask=lastm)
        return _i32(0)

    _fori(0, _NV, body, _i32(0))
    pltpu.sync_copy(kv, h_o.at[pl.ds(t * _i32(_CHUNK), _CHUNK)])
    pltpu.sync_copy(hist, hist_o.at[t])


@functools.partial(
    pl.kernel, mesh=_mesh, compiler_params=_scp,
    out_type=jax.ShapeDtypeStruct((_NW, _RADIX), jnp.int32),
    scratch_types=[pltpu.VMEM((_CHUNK,), jnp.int32),
                   pltpu.VMEM((_RADIX,), jnp.int32)],
)
def _k_hist12(keys_hbm, hist_o, kv, hist):
    t = _wid()
    pltpu.sync_copy(keys_hbm.at[pl.ds(t * _i32(_CHUNK), _CHUNK)], kv)

    def zb(k, _):
        hist[pl.ds(k * _i32(16), 16)] = jnp.zeros((16,), jnp.int32)
        return _i32(0)

    _fori(0, _RADIX // 16, zb, _i32(0))

    def body(j, _):
        v = kv[pl.ds(j * _i32(16), 16)]
        d = lax.shift_right_logical(v, _i32(12)) & _i32(_RADIX - 1)
        occ, lastm = plsc.scan_count(d)
        plsc.addupdate_scatter(hist.at[pl.ds(0, _RADIX)], [d], occ,
                               mask=lastm)
        return _i32(0)

    _fori(0, _NV, body, _i32(0))
    pltpu.sync_copy(hist, hist_o.at[t])


@functools.partial(
    pl.kernel, mesh=_mesh, compiler_params=_scp,
    out_type=(jax.ShapeDtypeStruct((_RADIX,), jnp.int32),
              jax.ShapeDtypeStruct((_NW, _RADIX), jnp.int32)),
    scratch_types=[pltpu.VMEM((_NW, 128), jnp.int32),
                   pltpu.VMEM((_NW, 128), jnp.int32),
                   pltpu.VMEM((128,), jnp.int32)],
)
def _k_scan(hist_hbm, tot_o, scan_o, hv, pv, totv):
    w = _wid()
    dbase = w * _i32(128)
    for t in range(_NW):
        pltpu.sync_copy(hist_hbm.at[_i32(t), pl.ds(dbase, 128)],
                        hv.at[_i32(t)])
    for seg in range(8):
        sl = pl.ds(seg * 16, 16)
        acc = jnp.zeros((16,), jnp.int32)
        for t in range(_NW):
            pv[_i32(t), sl] = acc
            acc = acc + hv[_i32(t), sl]
        totv[sl] = acc
    pltpu.sync_copy(totv, tot_o.at[pl.ds(dbase, 128)])
    for t in range(_NW):
        pltpu.sync_copy(pv.at[_i32(t)], scan_o.at[_i32(t), pl.ds(dbase, 128)])


def _make_perm(shift):
    @functools.partial(
        pl.kernel, mesh=_mesh, compiler_params=_scp,
        out_type=(jax.ShapeDtypeStruct((_N,), jnp.int32),
                  jax.ShapeDtypeStruct((_N,), jnp.int32)),
        scratch_types=[pltpu.VMEM((_CHUNK,), jnp.int32),
                       pltpu.VMEM((_CHUNK,), jnp.int32),
                       pltpu.VMEM((_RADIX,), jnp.int32),
                       pltpu.VMEM((_RADIX,), jnp.int32),
                       pltpu.VMEM((64, 128), jnp.int32),
                       pltpu.SemaphoreType.DMA],
    )
    def _k_perm(keys_hbm, vals_hbm, tot_hbm, scan_hbm, ko_o, vo_o,
                kv, vv, tv, noff, dst2, sem):
        t = _wid()
        pltpu.sync_copy(keys_hbm.at[pl.ds(t * _i32(_CHUNK), _CHUNK)], kv)
        pltpu.sync_copy(vals_hbm.at[pl.ds(t * _i32(_CHUNK), _CHUNK)], vv)
        pltpu.sync_copy(tot_hbm, tv)
        pltpu.sync_copy(scan_hbm.at[t], noff)

        def scan_body(k, carry):
            sl = pl.ds(k * _i32(16), 16)
            v = tv[sl]
            c = plsc.cumsum(v)
            noff[sl] = noff[sl] + (c - v) + carry
            return carry + c[15]

        _fori(0, _RADIX // 16, scan_body, _i32(0))

        io = _iota()

        def body(j, _):
            v = kv[pl.ds(j * _i32(16), 16)]
            d = lax.shift_right_logical(v, _i32(shift)) & _i32(_RADIX - 1)
            occ, lastm = plsc.scan_count(d)
            cur = plsc.load_gather(noff.at[pl.ds(0, _RADIX)], [d])
            dstv = cur + occ - _i32(1)
            plsc.store_scatter(noff.at[pl.ds(0, _RADIX)], [d], cur + occ,
                               mask=lastm)
            rows = jnp.zeros((16,), jnp.int32) + j // _i32(8)
            cols = (j % _i32(8)) * _i32(16) + io
            plsc.store_scatter(dst2.at[:, :], [rows, cols], dstv)
            return _i32(0)

        _fori(0, _NV, body, _i32(0))

        hs = []
        for c in range(64):
            hs.append(pltpu.async_copy(
                kv.at[pl.ds(c * 128, 128)], ko_o.at[dst2.at[_i32(c)]], sem))
            hs.append(pltpu.async_copy(
                vv.at[pl.ds(c * 128, 128)], vo_o.at[dst2.at[_i32(c)]], sem))
            if c % 8 == 7:
                for h in hs:
                    h.wait()
                hs = []

    return _k_perm


_perm0 = _make_perm(0)
_perm12 = _make_perm(12)


@functools.partial(
    pl.kernel, mesh=_mesh, compiler_params=_scp,
    out_type=(jax.ShapeDtypeStruct((_NW, _CHUNK), jnp.int32),
              jax.ShapeDtypeStruct((_NW, _CHUNK), jnp.int32),
              jax.ShapeDtypeStruct((_NW, 16), jnp.int32)),
    scratch_types=[pltpu.VMEM((_CHUNK,), jnp.int32),
                   pltpu.VMEM((_CHUNK,), jnp.int32),
                   pltpu.VMEM((_CHUNK + 16,), jnp.int32),
                   pltpu.VMEM((_CHUNK + 16,), jnp.int32),
                   pltpu.VMEM((16,), jnp.int32),
                   pltpu.VMEM((16,), jnp.int32)],
)
def _k_win(ks_hbm, vs_hbm, wi_o, wh_o, wcnt_o, kv, vv, wiv, whv, pb, t16):
    t = _wid()
    pltpu.sync_copy(ks_hbm.at[pl.ds(t * _i32(_CHUNK), _CHUNK)], kv)
    pltpu.sync_copy(vs_hbm.at[pl.ds(t * _i32(_CHUNK), _CHUNK)], vv)

    @pl.when(t > 0)
    def _():
        pltpu.sync_copy(ks_hbm.at[pl.ds(t * _i32(_CHUNK) - 16, 16)], pb)

    @pl.when(t == 0)
    def _():
        pb[...] = jnp.full((16,), -1, jnp.int32)

    io = _iota()
    carry0 = pb[...][15]

    def body(j, carry):
        prevlast, wcount = carry
        sl = pl.ds(j * _i32(16), 16)
        v = kv[sl]
        t16[...] = v
        shv = plsc.load_gather(t16.at[pl.ds(0, 16)],
                               [jnp.maximum(io - _i32(1), _i32(0))])
        shv = jnp.where(io == _i32(0), prevlast, shv)
        f = v != shv
        plsc.store_compressed(whv.at[pl.ds(wcount, 16)], v, mask=f)
        plsc.store_compressed(wiv.at[pl.ds(wcount, 16)], vv[sl], mask=f)
        pc = plsc.all_reduce_population_count(f)[0]
        return v[15], wcount + pc

    _, wcount = _fori(0, _NV, body, (carry0, _i32(0)))
    pltpu.sync_copy(wiv.at[pl.ds(0, _CHUNK)], wi_o.at[t])
    pltpu.sync_copy(whv.at[pl.ds(0, _CHUNK)], wh_o.at[t])
    t16[...] = jnp.zeros((16,), jnp.int32) + wcount
    pltpu.sync_copy(t16, wcnt_o.at[t])


@functools.partial(
    pl.kernel, mesh=_mesh, compiler_params=_scp,
    out_type=jax.ShapeDtypeStruct((_NW, 16), jnp.int32),
    scratch_types=[pltpu.VMEM((_CHUNK,), jnp.int32),
                   pltpu.VMEM((_CHUNK,), jnp.int32),
                   pltpu.VMEM((128,), jnp.int32),
                   pltpu.VMEM((128,), jnp.int32),
                   pltpu.VMEM((128,), jnp.int32),
                   pltpu.VMEM((128,), jnp.int32),
                   pltpu.VMEM((128,), jnp.int32),
                   pltpu.VMEM((128,), jnp.float32),
                   pltpu.VMEM((128,), jnp.float32),
                   pltpu.VMEM((128,), jnp.float32),
                   pltpu.VMEM((128,), jnp.float32),
                   pltpu.VMEM((128,), jnp.float32),
                   pltpu.VMEM((128,), jnp.float32),
                   pltpu.VMEM((128,), jnp.float32),
                   pltpu.VMEM((128,), jnp.float32),
                   pltpu.VMEM((128,), jnp.float32),
                   pltpu.VMEM((128,), jnp.int32),
                   pltpu.VMEM((128,), jnp.int32),
                   pltpu.VMEM((128,), jnp.int32),
                   pltpu.VMEM((_CHUNK + 16,), jnp.int32),
                   pltpu.VMEM((_CHUNK + 16,), jnp.float32),
                   pltpu.VMEM((_CHUNK + 16,), jnp.float32),
                   pltpu.VMEM((_CHUNK + 16,), jnp.float32),
                   pltpu.VMEM((16,), jnp.int32),
                   pltpu.VMEM((16,), jnp.float32),
                   pltpu.SemaphoreType.DMA],
)
def _k_updmask(wi_hbm, wh_hbm, wcnt_hbm, buf_hbm, npf_hbm, vcm_hbm,
               ptsf_hbm, colf_hbm, upd_ref, out_ref, wupd_o,
               wiv, whv, bidx, lov, sidx, wic, gidx, gx, gy, gz,
               px, py, pz, cxv, cyv, czv, gv,
               uidx, udat, csafe, ccx, ccy, ccz, i16, f16, sem):
    t = _wid()
    bsrc = buf_hbm
    pltpu.sync_copy(wcnt_hbm.at[t], i16)
    nw = i16[...][0]
    pltpu.sync_copy(wi_hbm.at[t], wiv)
    pltpu.sync_copy(wh_hbm.at[t], whv)
    io = _iota()
    nc = (nw + _i32(127)) // _i32(128)

    def chunk_body(c, ccnt):
        cb = c * _i32(128)

        def bi(j, _):
            sl = pl.ds(j * _i32(16), 16)
            wh = jnp.clip(whv[pl.ds(cb + j * _i32(16), 16)],
                          _i32(0), _i32(_B - 1))
            wival = jnp.clip(wiv[pl.ds(cb + j * _i32(16), 16)],
                             _i32(0), _i32(_N - 1))
            bidx[sl] = wh * _i32(2)
            wic[sl] = wival
            return _i32(0)

        _fori(0, 8, bi, _i32(0))
        pltpu.async_copy(bsrc.at[bidx], lov, sem).wait()

        def si(j, _):
            sl = pl.ds(j * _i32(16), 16)
            sidx[sl] = jnp.clip(lov[sl], _i32(0), _i32(_M - 1))
            return _i32(0)

        _fori(0, 8, si, _i32(0))
        pltpu.async_copy(vcm_hbm.at[sidx], gv, sem).wait()
        for c3, dst in ((0, gx), (1, gy), (2, gz)):
            def mk(j, _, c3=c3):
                sl = pl.ds(j * _i32(16), 16)
                gidx[sl] = sidx[sl] * _i32(3) + _i32(c3)
                return _i32(0)

            _fori(0, 8, mk, _i32(0))
            pltpu.async_copy(npf_hbm.at[gidx], dst, sem).wait()
        for c3, dst, dst2 in ((0, px, cxv), (1, py, cyv), (2, pz, czv)):
            def mk2(j, _, c3=c3):
                sl = pl.ds(j * _i32(16), 16)
                gidx[sl] = wic[sl] * _i32(3) + _i32(c3)
                return _i32(0)

            _fori(0, 8, mk2, _i32(0))
            pltpu.async_copy(ptsf_hbm.at[gidx], dst, sem).wait()
            pltpu.async_copy(colf_hbm.at[gidx], dst2, sem).wait()

        def cv(j, ccnt):
            sl = pl.ds(j * _i32(16), 16)
            valid = (cb + j * _i32(16) + io) < nw
            lo = lov[sl]
            dx = gx[sl] - px[sl]
            dy = gy[sl] - py[sl]
            dz = gz[sl] - pz[sl]
            d2 = (dx * dx + dy * dy) + dz * dz
            updv = ((lo == _i32(-1)) | (d2 > _f32(_THR)))
            updv = jnp.where(valid, updv.astype(jnp.int32), _i32(0))
            cmask = (lo > _i32(-1)) & (gv[sl] == _i32(0)) & valid
            wival = wic[sl]
            uidx[sl] = wival
            udat[sl] = updv
            plsc.store_compressed(csafe.at[pl.ds(ccnt, 16)], sidx[sl],
                                  mask=cmask)
            plsc.store_compressed(ccx.at[pl.ds(ccnt, 16)], cxv[sl], mask=cmask)
            plsc.store_compressed(ccy.at[pl.ds(ccnt, 16)], cyv[sl], mask=cmask)
            plsc.store_compressed(ccz.at[pl.ds(ccnt, 16)], czv[sl], mask=cmask)
            pc = plsc.all_reduce_population_count(cmask)[0]
            return ccnt + pc

        ccnt = _fori(0, 8, cv, ccnt)

        u0 = uidx[pl.ds(0, 16)][0]
        d0 = udat[pl.ds(0, 16)][0]

        def fixv(j, _):
            sl = pl.ds(j * _i32(16), 16)
            valid = (cb + j * _i32(16) + io) < nw
            uidx[sl] = jnp.where(valid, uidx[sl], u0)
            udat[sl] = jnp.where(valid, udat[sl], d0)
            return _i32(0)

        _fori(0, 8, fixv, _i32(0))
        pltpu.async_copy(udat, upd_ref.at[uidx], sem).wait()
        return ccnt

    ccnt = _fori(0, nc, chunk_body, _i32(0))
    ncv = (ccnt + _i32(15)) // _i32(16)

    def col_body(j, _):
        sl = pl.ds(j * _i32(16), 16)
        valid = (j * _i32(16) + io) < ccnt
        s = csafe[sl]
        s = jnp.where(valid, s, s[0])
        vx = ccx[sl]
        vy = ccy[sl]
        vz = ccz[sl]
        base = _i32(_CBASE) + s * _i32(3)
        i16[...] = base
        f16[...] = jnp.where(valid, vx, vx[0])
        pltpu.async_copy(f16, out_ref.at[i16], sem).wait()
        i16[...] = base + _i32(1)
        f16[...] = jnp.where(valid, vy, vy[0])
        pltpu.async_copy(f16, out_ref.at[i16], sem).wait()
        i16[...] = base + _i32(2)
        f16[...] = jnp.where(valid, vz, vz[0])
        pltpu.async_copy(f16, out_ref.at[i16], sem).wait()
        return _i32(0)

    _fori(0, ncv, col_body, _i32(0))
    i16[...] = jnp.zeros((16,), jnp.int32)
    pltpu.sync_copy(i16, wupd_o.at[t])


@functools.partial(
    pl.kernel, mesh=_mesh, compiler_params=_scp,
    out_type=jax.ShapeDtypeStruct((_NW, 16), jnp.int32),
    scratch_types=[pltpu.VMEM((_CHUNK,), jnp.int32),
                   pltpu.VMEM((16,), jnp.int32)],
)
def _k_cnt(upd_ref, cnt_o, uv, t16):
    t = _wid()
    pltpu.sync_copy(upd_ref.at[pl.ds(t * _i32(_CHUNK), _CHUNK)], uv)

    def body(j, s):
        v = uv[pl.ds(j * _i32(16), 16)]
        return s + plsc.cumsum(v)[15]

    s = _fori(0, _NV, body, _i32(0))
    t16[...] = jnp.zeros((16,), jnp.int32) + s
    pltpu.sync_copy(t16, cnt_o.at[t])


@functools.partial(
    pl.kernel, mesh=_mesh, compiler_params=_scp,
    out_type=jax.ShapeDtypeStruct((_NW, 16), jnp.int32),
    scratch_types=[pltpu.VMEM((512,), jnp.int32),
                   pltpu.VMEM((2048,), jnp.int32),
                   pltpu.VMEM((2048,), jnp.int32),
                   pltpu.VMEM((3 * 2048,), jnp.float32),
                   pltpu.VMEM((3 * 2048,), jnp.float32),
                   pltpu.VMEM((_CHUNK + 16,), jnp.int32),
                   pltpu.VMEM((_CHUNK + 16,), jnp.int32),
                   pltpu.VMEM((_CHUNK + 16,), jnp.float32),
                   pltpu.VMEM((_CHUNK + 16,), jnp.float32),
                   pltpu.VMEM((_CHUNK + 16,), jnp.float32),
                   pltpu.VMEM((_CHUNK + 16,), jnp.float32),
                   pltpu.VMEM((_CHUNK + 16,), jnp.float32),
                   pltpu.VMEM((_CHUNK + 16,), jnp.float32),
                   pltpu.VMEM((128,), jnp.int32),
                   pltpu.VMEM((128,), jnp.int32),
                   pltpu.VMEM((16,), jnp.int32),
                   pltpu.SemaphoreType.DMA],
)
def _k_scat(ucntf_hbm, hf_hbm, pts3_hbm, nrm3_hbm, upd_ref, out_ref,
            buf_ref, done_o,
            ucv, uv, hv, ptv, nmv,
            cslot, chh, csx, csy, csz, csnx, csny, csnz,
            idxb, zi128, t16, sem):
    t = _wid()
    bdst = buf_ref
    pltpu.sync_copy(ucntf_hbm, ucv)
    io = _iota()
    g1 = plsc.load_gather(ucv.at[pl.ds(0, 512)], [io * _i32(16)])
    g2 = plsc.load_gather(ucv.at[pl.ds(0, 512)], [io * _i32(16) + _i32(256)])
    base = (plsc.cumsum(jnp.where(io < t, g1, _i32(0)))[15]
            + plsc.cumsum(jnp.where(io + _i32(16) < t, g2, _i32(0)))[15])
    for k in range(8):
        zi128[pl.ds(k * 16, 16)] = jnp.zeros((16,), jnp.int32)

    def sub(sb, cnt):
        off = t * _i32(_CHUNK) + sb * _i32(2048)
        pltpu.sync_copy(upd_ref.at[pl.ds(off, 2048)], uv)
        pltpu.sync_copy(hf_hbm.at[pl.ds(off, 2048)], hv)
        pltpu.sync_copy(pts3_hbm.at[pl.ds(off * _i32(3), 3 * 2048)], ptv)
        pltpu.sync_copy(nrm3_hbm.at[pl.ds(off * _i32(3), 3 * 2048)], nmv)

        def vec(j, cnt):
            sl = pl.ds(j * _i32(16), 16)
            u = uv[sl]
            ub = u > _i32(0)
            pfx = plsc.cumsum(u)
            slot = _i32(_M) + base + cnt + pfx - _i32(1)
            lanes = (j * _i32(16) + io) * _i32(3)
            x = plsc.load_gather(ptv.at[pl.ds(0, 3 * 2048)], [lanes])
            y = plsc.load_gather(ptv.at[pl.ds(0, 3 * 2048)],
                                 [lanes + _i32(1)])
            z = plsc.load_gather(ptv.at[pl.ds(0, 3 * 2048)],
                                 [lanes + _i32(2)])
            nx = plsc.load_gather(nmv.at[pl.ds(0, 3 * 2048)], [lanes])
            ny = plsc.load_gather(nmv.at[pl.ds(0, 3 * 2048)],
                                  [lanes + _i32(1)])
            nz = plsc.load_gather(nmv.at[pl.ds(0, 3 * 2048)],
                                  [lanes + _i32(2)])
            s = nx * nx + ny * ny + nz * nz
            r = s * _rsqrt_sum(s) + _f32(1e-8)
            nx = nx / r
            ny = ny / r
            nz = nz / r
            plsc.store_compressed(cslot.at[pl.ds(cnt, 16)], slot, mask=ub)
            plsc.store_compressed(chh.at[pl.ds(cnt, 16)], hv[sl], mask=ub)
            plsc.store_compressed(csx.at[pl.ds(cnt, 16)], x, mask=ub)
            plsc.store_compressed(csy.at[pl.ds(cnt, 16)], y, mask=ub)
            plsc.store_compressed(csz.at[pl.ds(cnt, 16)], z, mask=ub)
            plsc.store_compressed(csnx.at[pl.ds(cnt, 16)], nx, mask=ub)
            plsc.store_compressed(csny.at[pl.ds(cnt, 16)], ny, mask=ub)
            plsc.store_compressed(csnz.at[pl.ds(cnt, 16)], nz, mask=ub)
            return cnt + pfx[15]

        return _fori(0, 128, vec, cnt)

    cnt = _fori(0, 4, sub, _i32(0))

    @pl.when(cnt > 0)
    def _():
        d_slot = cslot[pl.ds(0, 16)][0]
        d_h = chh[pl.ds(0, 16)][0]
        d_x = csx[pl.ds(0, 16)][0]
        d_y = csy[pl.ds(0, 16)][0]
        d_z = csz[pl.ds(0, 16)][0]
        d_nx = csnx[pl.ds(0, 16)][0]
        d_ny = csny[pl.ds(0, 16)][0]
        d_nz = csnz[pl.ds(0, 16)][0]
        padend = ((cnt + _i32(127)) // _i32(128)) * _i32(128)

        def fill(j, _):
            sl = pl.ds(j * _i32(16), 16)
            valid = (j * _i32(16) + io) < cnt
            cslot[sl] = jnp.where(valid, cslot[sl], d_slot)
            chh[sl] = jnp.where(valid, chh[sl], d_h)
            csx[sl] = jnp.where(valid, csx[sl], d_x)
            csy[sl] = jnp.where(valid, csy[sl], d_y)
            csz[sl] = jnp.where(valid, csz[sl], d_z)
            csnx[sl] = jnp.where(valid, csnx[sl], d_nx)
            csny[sl] = jnp.where(valid, csny[sl], d_ny)
            csnz[sl] = jnp.where(valid, csnz[sl], d_nz)
            return _i32(0)

        _fori(cnt // _i32(16), padend // _i32(16), fill, _i32(0))
        nch = (cnt + _i32(127)) // _i32(128)

        def sc(c, _):
            cb = c * _i32(128)
            srcs = (csx, csy, csz, csnx, csny, csnz)
            for c3 in range(3):
                def mkidx(j, _2, c3=c3):
                    sl = pl.ds(j * _i32(16), 16)
                    s = cslot[pl.ds(cb + j * _i32(16), 16)]
                    idxb[sl] = s * _i32(3) + _i32(c3)
                    return _i32(0)

                _fori(0, 8, mkidx, _i32(0))
                pltpu.async_copy(srcs[c3].at[pl.ds(cb, 128)],
                                 out_ref.at[idxb], sem).wait()

                def mkidx2(j, _2, c3=c3):
                    sl = pl.ds(j * _i32(16), 16)
                    s = cslot[pl.ds(cb + j * _i32(16), 16)]
                    idxb[sl] = _i32(_ORBASE) + s * _i32(3) + _i32(c3)
                    return _i32(0)

                _fori(0, 8, mkidx2, _i32(0))
                pltpu.async_copy(srcs[3 + c3].at[pl.ds(cb, 128)],
                                 out_ref.at[idxb], sem).wait()

            def mkidxb(j, _2):
                sl = pl.ds(j * _i32(16), 16)
                hh = chh[pl.ds(cb + j * _i32(16), 16)]
                idxb[sl] = hh * _i32(2)
                return _i32(0)

            _fori(0, 8, mkidxb, _i32(0))
            pltpu.async_copy(cslot.at[pl.ds(cb, 128)],
                             bdst.at[idxb], sem).wait()

            def mkidxb2(j, _2):
                sl = pl.ds(j * _i32(16), 16)
                hh = chh[pl.ds(cb + j * _i32(16), 16)]
                idxb[sl] = hh * _i32(2) + _i32(1)
                return _i32(0)

            _fori(0, 8, mkidxb2, _i32(0))
            pltpu.async_copy(zi128.at[pl.ds(0, 128)],
                             bdst.at[idxb], sem).wait()
            return _i32(0)

        _fori(0, nch, sc, _i32(0))

    t16[...] = jnp.zeros((16,), jnp.int32)
    pltpu.sync_copy(t16, done_o.at[t])


def kernel(points, colors, normals, buffer_pt_index, neural_points,
           point_colors, valid_color_mask, point_ts_update, travel_dist,
           cur_ts):
    ptsf = points.reshape(3 * _N)
    nrmf = normals.reshape(3 * _N)

    hf, hist1 = _k_hh(ptsf)
    tot1, scan1 = _k_scan(hist1)
    v0 = jnp.arange(_N, dtype=jnp.int32)
    k1, v1 = _perm0(hf, v0, tot1, scan1)
    hist2 = _k_hist12(k1)
    tot2, scan2 = _k_scan(hist2)
    k2, v2 = _perm12(k1, v1, tot2, scan2)
    wi, wh, wcnt = _k_win(k2, v2)

    out_r = jax.empty_ref(jax.ShapeDtypeStruct((_OUTLEN,), jnp.float32))
    upd_r = jax.empty_ref(jax.ShapeDtypeStruct((_N,), jnp.int32))
    _k_base(neural_points.reshape(3 * _M), point_colors.reshape(3 * _M),
            out_r, upd_r)

    bufflat = lax.bitcast_convert_type(buffer_pt_index,
                                       jnp.int32).reshape(2 * _B)
    _k_updmask(wi, wh, wcnt, bufflat, neural_points.reshape(3 * _M),
               valid_color_mask, ptsf, colors.reshape(3 * _N), upd_r, out_r)

    ucnt = _k_cnt(upd_r)

    bufcp = _tc2(bufflat.reshape(125, 1250, 128))
    buf_r = jax.new_ref(bufcp.reshape(2 * _B))
    _k_scat(ucnt.reshape(512), hf, ptsf, nrmf, upd_r, out_r, buf_r)

    out = out_r[...].reshape(2 * _CAP + _M, 3)
    buffer_new = lax.bitcast_convert_type(
        buf_r[...].reshape(_B, 2), jnp.int64)
    return out, buffer_new
